# baseline (speedup 1.0000x reference)
import functools

import jax
import jax.numpy as jnp
from jax.experimental import pallas as pl
from jax.experimental.pallas import tpu as pltpu


def _round_up(x, m):
    return ((x + m - 1) // m) * m


def _cdiv(a, b):
    return (a + b - 1) // b


_TB = 8


def _head_contig_kernel(hA_ref, hB_ref, hC_ref, w_ref, b_ref, o_ref,
                        acc_ref, *, inv_seq_len, seq_len):
    k = pl.program_id(1)
    half = _TB // 2
    quart = _TB // 4
    H = acc_ref.shape[-1]

    @pl.when(k == 0)
    def _():
        hA = hA_ref[0].astype(jnp.float32)
        acc_ref[:half, :] = jnp.sum(hA.reshape(half, seq_len, H), axis=1)

    @pl.when(k == 1)
    def _():
        hB = hB_ref[0].astype(jnp.float32)
        acc_ref[half:half + quart, :] = jnp.sum(
            hB.reshape(quart, seq_len, H), axis=1)

    @pl.when(k == 2)
    def _():
        hC = hC_ref[0].astype(jnp.float32)
        acc_ref[half + quart:, :] = jnp.sum(
            hC.reshape(quart, seq_len, H), axis=1)
        pooled = acc_ref[...] * inv_seq_len
        logits = jax.lax.dot_general(
            pooled, w_ref[...].astype(jnp.float32),
            (((1,), (1,)), ((), ())), preferred_element_type=jnp.float32)
        o_ref[...] = logits + b_ref[...].astype(jnp.float32)


def _head_strided_kernel(h_ref, w_ref, b_ref, o_ref, acc_ref, *,
                         inv_seq_len, seq_tail, nk):
    k = pl.program_id(1)

    @pl.when(k == 0)
    def _():
        acc_ref[...] = jnp.zeros_like(acc_ref)

    if seq_tail:
        @pl.when(k != nk - 1)
        def _():
            acc_ref[...] += jnp.sum(h_ref[...].astype(jnp.float32), axis=1)

        @pl.when(k == nk - 1)
        def _():
            h = h_ref[...].astype(jnp.float32)
            sidx = jax.lax.broadcasted_iota(jnp.int32, h.shape, 1)
            acc_ref[...] += jnp.sum(jnp.where(sidx < seq_tail, h, 0.0), axis=1)
    else:
        acc_ref[...] += jnp.sum(h_ref[...].astype(jnp.float32), axis=1)

    @pl.when(k == nk - 1)
    def _():
        pooled = acc_ref[...] * inv_seq_len
        logits = jax.lax.dot_general(
            pooled, w_ref[...].astype(jnp.float32),
            (((1,), (1,)), ((), ())), preferred_element_type=jnp.float32)
        o_ref[...] = logits + b_ref[...].astype(jnp.float32)


def kernel(hidden_state, weight, bias):
    B, S, H = hidden_state.shape
    L = weight.shape[0]
    h_itemsize = jnp.dtype(hidden_state.dtype).itemsize

    Lp = _round_up(max(L, 1), 128)
    w2 = weight
    b2 = bias.reshape(1, L)
    if Lp != L:
        w2 = jnp.pad(w2, ((0, Lp - L), (0, 0)))
        b2 = jnp.pad(b2, ((0, 0), (0, Lp - L)))
    fixed = H * Lp * 4 + 2 * Lp * 4 + 2 * 16 * Lp * 4 + 16 * H * 4

    big_bytes = (_TB // 2) * S * H * h_itemsize
    if B % _TB == 0 and B // _TB >= 2 and big_bytes <= (12 << 20):
        nb = B // _TB
        CA = (_TB // 2) * S
        CB = (_TB // 4) * S
        hv = hidden_state.reshape(nb, _TB * S, H)
        vmem_limit = int(min(2 * (big_bytes + big_bytes // 2 + big_bytes // 2)
                             + fixed + (8 << 20), 56 << 20))
        out = pl.pallas_call(
            functools.partial(_head_contig_kernel,
                              inv_seq_len=1.0 / S, seq_len=S),
            out_shape=jax.ShapeDtypeStruct((B, Lp), jnp.float32),
            grid_spec=pltpu.PrefetchScalarGridSpec(
                num_scalar_prefetch=0,
                grid=(nb, 3),
                in_specs=[
                    pl.BlockSpec((1, CA, H), lambda b, k: (b, 0, 0)),
                    pl.BlockSpec((1, CB, H), lambda b, k: (b, 2, 0)),
                    pl.BlockSpec((1, CB, H), lambda b, k: (b, 3, 0)),
                    pl.BlockSpec((Lp, H), lambda b, k: (0, 0),
                                 pipeline_mode=pl.Buffered(1)),
                    pl.BlockSpec((1, Lp), lambda b, k: (0, 0),
                                 pipeline_mode=pl.Buffered(1)),
                ],
                out_specs=pl.BlockSpec((_TB, Lp), lambda b, k: (b, 0)),
                scratch_shapes=[pltpu.VMEM((_TB, H), jnp.float32)],
            ),
            compiler_params=pltpu.CompilerParams(
                dimension_semantics=("parallel", "arbitrary"),
                vmem_limit_bytes=vmem_limit),
        )(hv, hv, hv, w2, b2)
        return out[:, :L]

    TB = min(16, _round_up(B, 8))
    nb = _cdiv(B, TB)
    TS = max(8, ((6 << 20) // max(1, TB * H * h_itemsize)) // 8 * 8)
    TS = min(TS, _round_up(S, 8))
    nk = _cdiv(S, TS)
    seq_tail = S - (nk - 1) * TS
    if seq_tail == TS:
        seq_tail = 0
    blk = TB * TS * H * h_itemsize
    vmem_limit = int(min(2 * blk + fixed + (8 << 20), 56 << 20))
    out = pl.pallas_call(
        functools.partial(_head_strided_kernel, inv_seq_len=1.0 / S,
                          seq_tail=int(seq_tail), nk=nk),
        out_shape=jax.ShapeDtypeStruct((nb * TB, Lp), jnp.float32),
        grid_spec=pltpu.PrefetchScalarGridSpec(
            num_scalar_prefetch=0,
            grid=(nb, nk),
            in_specs=[
                pl.BlockSpec((TB, TS, H), lambda b, k: (b, k, 0)),
                pl.BlockSpec((Lp, H), lambda b, k: (0, 0),
                             pipeline_mode=pl.Buffered(1)),
                pl.BlockSpec((1, Lp), lambda b, k: (0, 0),
                             pipeline_mode=pl.Buffered(1)),
            ],
            out_specs=pl.BlockSpec((TB, Lp), lambda b, k: (b, 0)),
            scratch_shapes=[pltpu.VMEM((TB, H), jnp.float32)],
        ),
        compiler_params=pltpu.CompilerParams(
            dimension_semantics=("parallel", "arbitrary"),
            vmem_limit_bytes=vmem_limit),
    )(hidden_state, w2, b2)
    return out[:B, :L]


# speedup vs baseline: 1.3477x; 1.3477x over previous
import functools

import jax
import jax.numpy as jnp
from jax.experimental import pallas as pl
from jax.experimental.pallas import tpu as pltpu


def _round_up(x, m):
    return ((x + m - 1) // m) * m


def _cdiv(a, b):
    return (a + b - 1) // b


_TB = 8


def _head_contig_kernel(h_ref, w_ref, b_ref, o_ref, acc_ref, *,
                        inv_seq_len, seq_len):
    k = pl.program_id(1)
    half = _TB // 2
    h = h_ref[0, 0].astype(jnp.float32)
    s = jnp.sum(h.reshape(half, seq_len, h.shape[-1]), axis=1)

    @pl.when(k == 0)
    def _():
        acc_ref[:half, :] = s

    @pl.when(k == 1)
    def _():
        acc_ref[half:, :] = s
        pooled = acc_ref[...] * inv_seq_len
        logits = jax.lax.dot_general(
            pooled, w_ref[...].astype(jnp.float32),
            (((1,), (1,)), ((), ())), preferred_element_type=jnp.float32)
        o_ref[...] = logits + b_ref[...].astype(jnp.float32)


def _head_strided_kernel(h_ref, w_ref, b_ref, o_ref, acc_ref, *,
                         inv_seq_len, seq_tail, nk):
    k = pl.program_id(1)

    @pl.when(k == 0)
    def _():
        acc_ref[...] = jnp.zeros_like(acc_ref)

    if seq_tail:
        @pl.when(k != nk - 1)
        def _():
            acc_ref[...] += jnp.sum(h_ref[...].astype(jnp.float32), axis=1)

        @pl.when(k == nk - 1)
        def _():
            h = h_ref[...].astype(jnp.float32)
            sidx = jax.lax.broadcasted_iota(jnp.int32, h.shape, 1)
            acc_ref[...] += jnp.sum(jnp.where(sidx < seq_tail, h, 0.0), axis=1)
    else:
        acc_ref[...] += jnp.sum(h_ref[...].astype(jnp.float32), axis=1)

    @pl.when(k == nk - 1)
    def _():
        pooled = acc_ref[...] * inv_seq_len
        logits = jax.lax.dot_general(
            pooled, w_ref[...].astype(jnp.float32),
            (((1,), (1,)), ((), ())), preferred_element_type=jnp.float32)
        o_ref[...] = logits + b_ref[...].astype(jnp.float32)


def kernel(hidden_state, weight, bias):
    B, S, H = hidden_state.shape
    L = weight.shape[0]
    h_itemsize = jnp.dtype(hidden_state.dtype).itemsize

    Lp = _round_up(max(L, 1), 128)
    w2 = weight
    b2 = bias.reshape(1, L)
    if Lp != L:
        w2 = jnp.pad(w2, ((0, Lp - L), (0, 0)))
        b2 = jnp.pad(b2, ((0, 0), (0, Lp - L)))
    fixed = H * Lp * 4 + 2 * Lp * 4 + 2 * 16 * Lp * 4 + 16 * H * 4

    chunk_bytes = (_TB // 2) * S * H * h_itemsize
    if B % _TB == 0 and B // _TB >= 2 and chunk_bytes <= (12 << 20):
        nb = B // _TB
        C = (_TB // 2) * S
        hv = hidden_state.reshape(nb, 2, C, H)
        vmem_limit = int(min(2 * chunk_bytes + fixed + (8 << 20), 56 << 20))
        out = pl.pallas_call(
            functools.partial(_head_contig_kernel,
                              inv_seq_len=1.0 / S, seq_len=S),
            out_shape=jax.ShapeDtypeStruct((B, Lp), jnp.float32),
            grid_spec=pltpu.PrefetchScalarGridSpec(
                num_scalar_prefetch=0,
                grid=(nb, 2),
                in_specs=[
                    pl.BlockSpec((1, 1, C, H), lambda b, k: (b, k, 0, 0)),
                    pl.BlockSpec((Lp, H), lambda b, k: (0, 0),
                                 pipeline_mode=pl.Buffered(1)),
                    pl.BlockSpec((1, Lp), lambda b, k: (0, 0),
                                 pipeline_mode=pl.Buffered(1)),
                ],
                out_specs=pl.BlockSpec((_TB, Lp), lambda b, k: (b, 0)),
                scratch_shapes=[pltpu.VMEM((_TB, H), jnp.float32)],
            ),
            compiler_params=pltpu.CompilerParams(
                dimension_semantics=("parallel", "arbitrary"),
                vmem_limit_bytes=vmem_limit),
        )(hv, w2, b2)
        return out[:, :L]

    TB = min(16, _round_up(B, 8))
    nb = _cdiv(B, TB)
    TS = max(8, ((6 << 20) // max(1, TB * H * h_itemsize)) // 8 * 8)
    TS = min(TS, _round_up(S, 8))
    nk = _cdiv(S, TS)
    seq_tail = S - (nk - 1) * TS
    if seq_tail == TS:
        seq_tail = 0
    blk = TB * TS * H * h_itemsize
    vmem_limit = int(min(2 * blk + fixed + (8 << 20), 56 << 20))
    out = pl.pallas_call(
        functools.partial(_head_strided_kernel, inv_seq_len=1.0 / S,
                          seq_tail=int(seq_tail), nk=nk),
        out_shape=jax.ShapeDtypeStruct((nb * TB, Lp), jnp.float32),
        grid_spec=pltpu.PrefetchScalarGridSpec(
            num_scalar_prefetch=0,
            grid=(nb, nk),
            in_specs=[
                pl.BlockSpec((TB, TS, H), lambda b, k: (b, k, 0)),
                pl.BlockSpec((Lp, H), lambda b, k: (0, 0),
                             pipeline_mode=pl.Buffered(1)),
                pl.BlockSpec((1, Lp), lambda b, k: (0, 0),
                             pipeline_mode=pl.Buffered(1)),
            ],
            out_specs=pl.BlockSpec((TB, Lp), lambda b, k: (b, 0)),
            scratch_shapes=[pltpu.VMEM((TB, H), jnp.float32)],
        ),
        compiler_params=pltpu.CompilerParams(
            dimension_semantics=("parallel", "arbitrary"),
            vmem_limit_bytes=vmem_limit),
    )(hidden_state, w2, b2)
    return out[:B, :L]
